# baseline (speedup 1.0000x reference)
import functools
import math

import jax
import jax.numpy as jnp
from jax import lax
from jax.experimental import pallas as pl
from jax.experimental.pallas import tpu as pltpu
from jax.experimental.pallas import tpu_sc as plsc

_N = 512
_V = 8
_LANES = 16
_W = 2 * _N

_T = tuple(math.ceil(16.0 * 2.0 ** (k / 16.0)) for k in range(1, 16))


def _sc_bias(weight32):
    info = plsc.get_sparse_core_info()
    nc, ns = info.num_cores, info.num_subcores
    nw = nc * ns
    rows_per_w = _N // nw
    mesh = plsc.VectorSubcoreMesh(core_axis_name="c", subcore_axis_name="s")

    @functools.partial(
        pl.kernel,
        mesh=mesh,
        out_type=jax.ShapeDtypeStruct((_V, _N, _V * _N), jnp.float32),
        scratch_types=[
            pltpu.VMEM((32,), jnp.float32),
            pltpu.VMEM((_W,), jnp.float32),
            pltpu.VMEM((rows_per_w, _N), jnp.float32),
            pltpu.SemaphoreType.DMA,
        ],
    )
    def run(wt_hbm, out_hbm, wt_vm, win, buf, sem):
        wid = lax.axis_index("s") * nc + lax.axis_index("c")
        row0 = wid * rows_per_w
        pltpu.sync_copy(wt_hbm, wt_vm)
        lanes = lax.iota(jnp.int32, _LANES)

        wlo = wt_vm[pl.ds(0, _LANES)]
        whi = wt_vm[pl.ds(_LANES, _LANES)]
        w31 = jnp.full((_LANES,), whi[15], jnp.float32)

        def fill31(c, carry):
            win[pl.ds(c * _LANES, _LANES)] = w31
            return carry

        lax.fori_loop(0, (_N - 32) // _LANES, fill31, 0)
        assert [sum(1 for t in _T if 31 - l >= t) for l in range(_LANES)] == [
            15 - l + (1 if 4 <= l <= 11 else 0) for l in range(_LANES)
        ]
        plateau = jnp.logical_and(lanes >= 4, lanes <= 11)
        idx = 15 - lanes + jnp.where(plateau, 1, 0)
        win[pl.ds(_N - 32, _LANES)] = jnp.take(whi, idx)
        win[pl.ds(_N - 16, _LANES)] = lax.rev(wlo, (0,))
        w0 = jnp.full((_LANES,), wlo[0], jnp.float32)

        def fill0(c, carry):
            win[pl.ds(c * _LANES, _LANES)] = w0
            return carry

        lax.fori_loop(_N // _LANES, _W // _LANES, fill0, 0)

        g_rows = 8
        n_groups = rows_per_w // g_rows

        def build_one(t, carry):
            off = (_N - 1) - (row0 + t)

            def chunk(q, c2):
                buf[t, pl.ds(q * _LANES, _LANES)] = win[
                    pl.ds(off + q * _LANES, _LANES)
                ]
                return c2

            lax.fori_loop(0, _N // _LANES, chunk, 0)
            return carry

        def drain_group():
            def drain(q, carry):
                pltpu.make_async_copy(
                    buf.at[pl.ds(0, g_rows), :],
                    out_hbm.at[0, pl.ds(row0, g_rows), pl.ds(0, _N)],
                    sem,
                ).wait()
                return carry

            lax.fori_loop(0, _V * _V, drain, 0)

        def group(g, carry):
            lax.fori_loop(g * g_rows, (g + 1) * g_rows, build_one, 0)

            def issue(q, c2):
                k = q // _V
                c = q - k * _V
                pltpu.async_copy(
                    buf.at[pl.ds(g * g_rows, g_rows), :],
                    out_hbm.at[
                        k,
                        pl.ds(row0 + g * g_rows, g_rows),
                        pl.ds(c * _N, _N),
                    ],
                    sem,
                )
                return c2

            lax.fori_loop(0, _V * _V, issue, 0)

            @pl.when(g >= 2)
            def _():
                drain_group()

            return carry

        lax.fori_loop(0, n_groups, group, 0)
        for _ in range(min(n_groups, 2)):
            drain_group()

    return run(weight32)


def kernel(n_vars, n_tokens, weight):
    del n_vars, n_tokens
    out = _sc_bias(weight.reshape(32))
    return out.reshape(1, 1, _V * _N, _V * _N)
